# Initial kernel scaffold; baseline (speedup 1.0000x reference)
import jax
import jax.numpy as jnp
from jax.experimental import pallas as pl


def kernel(x, edge_index, edge_weight, l0_W1, l0_b1, l0_W2, l0_b2, l0_gamma, l0_beta, l1_W1, l1_b1, l1_W2, l1_b2, l1_gamma, l1_beta, Wm1, bm1, Wm2, bm2):
    raise NotImplementedError("write your pallas kernel here")


# speedup vs baseline: 4.5640x; 4.5640x over previous
import functools
import jax
import jax.numpy as jnp
from jax import lax
from jax.experimental import pallas as pl
from jax.experimental.pallas import tpu as pltpu
from jax.experimental.pallas import tpu_sc as plsc

N = 10000
E = 320000
D = 128
L = 16
NC = 2
NS = 16
NW = NC * NS
EPW = E // NW
CHUNK = 200
NCHUNK = EPW // CHUNK
RPT = 624
SPAN = N - (NS - 1) * RPT


def _sc_agg_body(h_hbm, src_hbm, dst_hbm, ew_hbm, out_hbm,
                 srcv, dstv, ewv, rows, acc, sem):
    c = lax.axis_index("c")
    s = lax.axis_index("s")
    wid = c * NS + s

    def _zrow(j, _):
        for r in range(D // L):
            rows[j, pl.ds(r * L, L)] = jnp.zeros((L,), jnp.float32)
        return 0
    lax.fori_loop(0, CHUNK, _zrow, 0)
    base_rows = s * RPT
    off = 0
    while off < SPAN:
        n = min(CHUNK, SPAN - off)
        pltpu.sync_copy(rows.at[pl.ds(0, n)], acc.at[pl.ds(base_rows + off, n)])
        off += n
    plsc.subcore_barrier()

    def _chunk(i, _):
        ebase = wid * EPW + i * CHUNK
        pltpu.sync_copy(src_hbm.at[pl.ds(ebase, CHUNK)], srcv)
        pltpu.sync_copy(dst_hbm.at[pl.ds(ebase, CHUNK)], dstv)
        pltpu.sync_copy(ew_hbm.at[pl.ds(ebase, CHUNK)], ewv)
        pltpu.async_copy(h_hbm.at[srcv], rows, sem).wait()

        def _scale(j, _):
            w = plsc.load_gather(ewv, [jnp.broadcast_to(j, (L,)).astype(jnp.int32)])
            for r in range(D // L):
                rows[j, pl.ds(r * L, L)] = rows[j, pl.ds(r * L, L)] * w
            return 0
        lax.fori_loop(0, CHUNK, _scale, 0)
        pltpu.sync_copy(rows, acc.at[dstv], add=True)
        return 0
    lax.fori_loop(0, NCHUNK, _chunk, 0)
    plsc.subcore_barrier()

    pltpu.sync_copy(acc.at[pl.ds(base_rows, SPAN)], out_hbm.at[c, pl.ds(base_rows, SPAN)])


@jax.jit
def _sc_aggregate(h, src, dst, ew):
    mesh = plsc.VectorSubcoreMesh(core_axis_name="c", subcore_axis_name="s")
    return pl.kernel(
        _sc_agg_body,
        out_type=jax.ShapeDtypeStruct((NC, N, D), jnp.float32),
        mesh=mesh,
        compiler_params=pltpu.CompilerParams(needs_layout_passes=False),
        scratch_types=[
            pltpu.VMEM((CHUNK,), jnp.int32),
            pltpu.VMEM((CHUNK,), jnp.int32),
            pltpu.VMEM((CHUNK,), jnp.float32),
            pltpu.VMEM((CHUNK, D), jnp.float32),
            pltpu.VMEM_SHARED((N, D), jnp.float32),
            pltpu.SemaphoreType.DMA,
        ],
    )(h, src, dst, ew)


BLK = 1000
NBLK = N // BLK


def _mlp_body(h_ref, a0_ref, a1_ref, w1_ref, b1_ref, w2_ref, b2_ref,
              z_ref, st_ref):
    z = h_ref[...] + a0_ref[...] + a1_ref[...]
    y = jnp.maximum(
        jnp.dot(z, w1_ref[...], preferred_element_type=jnp.float32) + b1_ref[...], 0.0)
    z2 = jnp.dot(y, w2_ref[...], preferred_element_type=jnp.float32) + b2_ref[...]
    z_ref[...] = z2
    s1 = jnp.sum(z2, axis=0, keepdims=True)
    s2 = jnp.sum(z2 * z2, axis=0, keepdims=True)
    st = jnp.concatenate([s1, s2, jnp.zeros((6, D), jnp.float32)], axis=0)

    @pl.when(pl.program_id(0) == 0)
    def _():
        st_ref[...] = jnp.zeros_like(st_ref)

    st_ref[...] += st


def _bn_body(z_ref, st_ref, g_ref, b_ref, o_ref):
    s = st_ref[...]
    mean = s[0:1] * (1.0 / N)
    var = s[1:2] * (1.0 / N) - mean * mean
    inv = lax.rsqrt(var + 1e-3)
    o_ref[...] = jnp.maximum(
        g_ref[...] * (z_ref[...] - mean) * inv + b_ref[...], 0.0)


def _head_body(h_ref, wm1_ref, bm1_ref, wm2_ref, bm2_ref, o_ref):
    y = jnp.maximum(
        jnp.dot(h_ref[...], wm1_ref[...], preferred_element_type=jnp.float32)
        + bm1_ref[...], 0.0)
    o_ref[...] = jnp.dot(y, wm2_ref[...], preferred_element_type=jnp.float32) + bm2_ref[...]


def _row_spec(width):
    return pl.BlockSpec((BLK, width), lambda i: (i, 0))


def _full_spec(shape):
    return pl.BlockSpec(shape, lambda i: tuple(0 for _ in shape))


@jax.jit
def _tc_mlp(h, a0, a1, w1, b1, w2, b2, gamma, beta):
    z, st = pl.pallas_call(
        _mlp_body,
        grid=(NBLK,),
        in_specs=[
            _row_spec(D), _row_spec(D), _row_spec(D),
            _full_spec((D, D)), _full_spec((1, D)),
            _full_spec((D, D)), _full_spec((1, D)),
        ],
        out_specs=[_row_spec(D), _full_spec((8, D))],
        out_shape=[
            jax.ShapeDtypeStruct((N, D), jnp.float32),
            jax.ShapeDtypeStruct((8, D), jnp.float32),
        ],
    )(h, a0, a1, w1, b1.reshape(1, D), w2, b2.reshape(1, D))
    return pl.pallas_call(
        _bn_body,
        grid=(NBLK,),
        in_specs=[
            _row_spec(D), _full_spec((8, D)),
            _full_spec((1, D)), _full_spec((1, D)),
        ],
        out_specs=_row_spec(D),
        out_shape=jax.ShapeDtypeStruct((N, D), jnp.float32),
    )(z, st, gamma.reshape(1, D), beta.reshape(1, D))


@jax.jit
def _tc_head(h, wm1, bm1, wm2, bm2):
    wm2p = jnp.zeros((256, 128), jnp.float32).at[:, :wm2.shape[1]].set(wm2)
    bm2p = jnp.zeros((1, 128), jnp.float32).at[:, :wm2.shape[1]].set(bm2.reshape(1, -1))
    out = pl.pallas_call(
        _head_body,
        grid=(NBLK,),
        in_specs=[
            _row_spec(D), _full_spec((D, 256)), _full_spec((1, 256)),
            _full_spec((256, 128)), _full_spec((1, 128)),
        ],
        out_specs=_row_spec(128),
        out_shape=jax.ShapeDtypeStruct((N, 128), jnp.float32),
    )(h, wm1, bm1.reshape(1, 256), wm2p, bm2p)
    return out[:, :wm2.shape[1]]


def kernel(x, edge_index, edge_weight, l0_W1, l0_b1, l0_W2, l0_b2, l0_gamma,
           l0_beta, l1_W1, l1_b1, l1_W2, l1_b2, l1_gamma, l1_beta,
           Wm1, bm1, Wm2, bm2):
    src = edge_index[0]
    dst = edge_index[1]
    h = x
    agg = _sc_aggregate(h, src, dst, edge_weight)
    h = _tc_mlp(h, agg[0], agg[1], l0_W1, l0_b1, l0_W2, l0_b2, l0_gamma, l0_beta)
    agg = _sc_aggregate(h, src, dst, edge_weight)
    h = _tc_mlp(h, agg[0], agg[1], l1_W1, l1_b1, l1_W2, l1_b2, l1_gamma, l1_beta)
    return _tc_head(h, Wm1, bm1, Wm2, bm2)


# speedup vs baseline: 7.2931x; 1.5980x over previous
import functools
import jax
import jax.numpy as jnp
from jax import lax
from jax.experimental import pallas as pl
from jax.experimental.pallas import tpu as pltpu
from jax.experimental.pallas import tpu_sc as plsc

N = 10000
E = 320000
D = 128
L = 16
NC = 2
NS = 16
NW = NC * NS
EPW = E // NW
CHUNK = 80
NCHUNK = EPW // CHUNK
NBUF = 3
FULLK = 41
RPT = 624
SPAN = N - (NS - 1) * RPT


def _sc_agg_body(h_hbm, src_hbm, dst_hbm, ew_hbm, out_hbm,
                 srcv0, srcv1, srcv2, dstv0, dstv1, dstv2,
                 ewv0, ewv1, ewv2, rows0, rows1, rows2, acc,
                 isem0, isem1, isem2, gsem0, gsem1, gsem2):
    srcv = (srcv0, srcv1, srcv2)
    dstv = (dstv0, dstv1, dstv2)
    ewv = (ewv0, ewv1, ewv2)
    rows = (rows0, rows1, rows2)
    isem = (isem0, isem1, isem2)
    gsem = (gsem0, gsem1, gsem2)
    c = lax.axis_index("c")
    s = lax.axis_index("s")
    wid = c * NS + s
    e0 = wid * EPW

    @plsc.parallel_loop(0, CHUNK, unroll=8)
    def _z(j):
        for r in range(D // L):
            rows[0][j, pl.ds(r * L, L)] = jnp.zeros((L,), jnp.float32)
    base_rows = s * RPT
    for m in range(SPAN // CHUNK):
        pltpu.sync_copy(rows[0], acc.at[pl.ds(base_rows + m * CHUNK, CHUNK)])
    plsc.subcore_barrier()

    def issue_idx(g, b):
        eb = e0 + g * CHUNK
        pltpu.async_copy(src_hbm.at[pl.ds(eb, CHUNK)], srcv[b], isem[b])
        pltpu.async_copy(dst_hbm.at[pl.ds(eb, CHUNK)], dstv[b], isem[b])
        pltpu.async_copy(ew_hbm.at[pl.ds(eb, CHUNK)], ewv[b], isem[b])

    def wait_idx(b):
        pltpu.make_async_copy(src_hbm.at[pl.ds(0, CHUNK)], srcv[b], isem[b]).wait()
        pltpu.make_async_copy(dst_hbm.at[pl.ds(0, CHUNK)], dstv[b], isem[b]).wait()
        pltpu.make_async_copy(ew_hbm.at[pl.ds(0, CHUNK)], ewv[b], isem[b]).wait()

    def issue_gather(b):
        pltpu.async_copy(h_hbm.at[srcv[b]], rows[b], gsem[b])

    def wait_gather(b):
        pltpu.make_async_copy(h_hbm.at[srcv[b]], rows[b], gsem[b]).wait()

    def do_chunk(g, b, pf_gather, pf_idx):
        b2 = (b + 2) % NBUF
        if pf_gather:
            wait_idx(b2)
            issue_gather(b2)
        wait_gather(b)

        @plsc.parallel_loop(0, CHUNK, unroll=8)
        def _scale(j):
            w = plsc.load_gather(ewv[b], [jnp.broadcast_to(j, (L,))])
            for r in range(D // L):
                rows[b][j, pl.ds(r * L, L)] = rows[b][j, pl.ds(r * L, L)] * w

        pltpu.sync_copy(rows[b], acc.at[dstv[b]], add=True)
        if pf_idx is not None:
            @pl.when(pf_idx)
            def _():
                issue_idx(g + NBUF, b)

    for b in range(NBUF):
        issue_idx(jnp.int32(b), b)
    for b in range(2):
        wait_idx(b)
        issue_gather(b)

    def _iter(k, _):
        g = k * NBUF
        do_chunk(g, 0, True, jnp.bool_(True))
        do_chunk(g + 1, 1, True, jnp.bool_(True))
        do_chunk(g + 2, 2, True, k < FULLK - 1)
        return 0
    lax.fori_loop(0, FULLK, _iter, 0)

    do_chunk(jnp.int32(FULLK * NBUF), 0, False, None)
    do_chunk(jnp.int32(FULLK * NBUF + 1), 1, False, None)
    plsc.subcore_barrier()

    pltpu.sync_copy(acc.at[pl.ds(base_rows, SPAN)], out_hbm.at[c, pl.ds(base_rows, SPAN)])


@jax.jit
def _sc_aggregate(h, src, dst, ew):
    mesh = plsc.VectorSubcoreMesh(core_axis_name="c", subcore_axis_name="s")
    return pl.kernel(
        _sc_agg_body,
        out_type=jax.ShapeDtypeStruct((NC, N, D), jnp.float32),
        mesh=mesh,
        compiler_params=pltpu.CompilerParams(needs_layout_passes=False),
        scratch_types=(
            [pltpu.VMEM((CHUNK,), jnp.int32)] * (2 * NBUF)
            + [pltpu.VMEM((CHUNK,), jnp.float32)] * NBUF
            + [pltpu.VMEM((CHUNK, D), jnp.float32)] * NBUF
            + [pltpu.VMEM_SHARED((N, D), jnp.float32)]
            + [pltpu.SemaphoreType.DMA] * (2 * NBUF)
        ),
    )(h, src, dst, ew)


BLK = 1000
NBLK = N // BLK


def _mlp_body(h_ref, a0_ref, a1_ref, w1_ref, b1_ref, w2_ref, b2_ref,
              z_ref, st_ref):
    z = h_ref[...] + a0_ref[...] + a1_ref[...]
    y = jnp.maximum(
        jnp.dot(z, w1_ref[...], preferred_element_type=jnp.float32) + b1_ref[...], 0.0)
    z2 = jnp.dot(y, w2_ref[...], preferred_element_type=jnp.float32) + b2_ref[...]
    z_ref[...] = z2
    s1 = jnp.sum(z2, axis=0, keepdims=True)
    s2 = jnp.sum(z2 * z2, axis=0, keepdims=True)
    st = jnp.concatenate([s1, s2, jnp.zeros((6, D), jnp.float32)], axis=0)

    @pl.when(pl.program_id(0) == 0)
    def _():
        st_ref[...] = jnp.zeros_like(st_ref)

    st_ref[...] += st


def _bn_body(z_ref, st_ref, g_ref, b_ref, o_ref):
    s = st_ref[...]
    mean = s[0:1] * (1.0 / N)
    var = s[1:2] * (1.0 / N) - mean * mean
    inv = lax.rsqrt(var + 1e-3)
    o_ref[...] = jnp.maximum(
        g_ref[...] * (z_ref[...] - mean) * inv + b_ref[...], 0.0)


def _head_body(h_ref, wm1_ref, bm1_ref, wm2_ref, bm2_ref, o_ref):
    y = jnp.maximum(
        jnp.dot(h_ref[...], wm1_ref[...], preferred_element_type=jnp.float32)
        + bm1_ref[...], 0.0)
    o_ref[...] = jnp.dot(y, wm2_ref[...], preferred_element_type=jnp.float32) + bm2_ref[...]


def _row_spec(width):
    return pl.BlockSpec((BLK, width), lambda i: (i, 0))


def _full_spec(shape):
    return pl.BlockSpec(shape, lambda i: tuple(0 for _ in shape))


@jax.jit
def _tc_mlp(h, a0, a1, w1, b1, w2, b2, gamma, beta):
    z, st = pl.pallas_call(
        _mlp_body,
        grid=(NBLK,),
        in_specs=[
            _row_spec(D), _row_spec(D), _row_spec(D),
            _full_spec((D, D)), _full_spec((1, D)),
            _full_spec((D, D)), _full_spec((1, D)),
        ],
        out_specs=[_row_spec(D), _full_spec((8, D))],
        out_shape=[
            jax.ShapeDtypeStruct((N, D), jnp.float32),
            jax.ShapeDtypeStruct((8, D), jnp.float32),
        ],
    )(h, a0, a1, w1, b1.reshape(1, D), w2, b2.reshape(1, D))
    return pl.pallas_call(
        _bn_body,
        grid=(NBLK,),
        in_specs=[
            _row_spec(D), _full_spec((8, D)),
            _full_spec((1, D)), _full_spec((1, D)),
        ],
        out_specs=_row_spec(D),
        out_shape=jax.ShapeDtypeStruct((N, D), jnp.float32),
    )(z, st, gamma.reshape(1, D), beta.reshape(1, D))


@jax.jit
def _tc_head(h, wm1, bm1, wm2, bm2):
    wm2p = jnp.zeros((256, 128), jnp.float32).at[:, :wm2.shape[1]].set(wm2)
    bm2p = jnp.zeros((1, 128), jnp.float32).at[:, :wm2.shape[1]].set(bm2.reshape(1, -1))
    out = pl.pallas_call(
        _head_body,
        grid=(NBLK,),
        in_specs=[
            _row_spec(D), _full_spec((D, 256)), _full_spec((1, 256)),
            _full_spec((256, 128)), _full_spec((1, 128)),
        ],
        out_specs=_row_spec(128),
        out_shape=jax.ShapeDtypeStruct((N, 128), jnp.float32),
    )(h, wm1, bm1.reshape(1, 256), wm2p, bm2p)
    return out[:, :wm2.shape[1]]


def kernel(x, edge_index, edge_weight, l0_W1, l0_b1, l0_W2, l0_b2, l0_gamma,
           l0_beta, l1_W1, l1_b1, l1_W2, l1_b2, l1_gamma, l1_beta,
           Wm1, bm1, Wm2, bm2):
    src = edge_index[0]
    dst = edge_index[1]
    h = x
    agg = _sc_aggregate(h, src, dst, edge_weight)
    h = _tc_mlp(h, agg[0], agg[1], l0_W1, l0_b1, l0_W2, l0_b2, l0_gamma, l0_beta)
    agg = _sc_aggregate(h, src, dst, edge_weight)
    h = _tc_mlp(h, agg[0], agg[1], l1_W1, l1_b1, l1_W2, l1_b2, l1_gamma, l1_beta)
    return _tc_head(h, Wm1, bm1, Wm2, bm2)


# speedup vs baseline: 9.5790x; 1.3134x over previous
import functools
import jax
import jax.numpy as jnp
from jax import lax
from jax.experimental import pallas as pl
from jax.experimental.pallas import tpu as pltpu
from jax.experimental.pallas import tpu_sc as plsc

N = 10000
E = 320000
D = 128
L = 16
NC = 2
NS = 16
NW = NC * NS
EPW = E // NW
CHUNK = 40
NCHUNK = EPW // CHUNK
NBUF = 5
PAIRS = 5
PAIR_E = EPW // PAIRS
PAIR_C = NCHUNK // PAIRS
GRPS = PAIR_C // NBUF
RPT = 624
SPAN = N - (NS - 1) * RPT


def _sc_agg_body(h_hbm, src_hbm, dst_hbm, ew_hbm, out0_hbm, out1_hbm,
                 srcp0, srcp1, ewp0, ewp1,
                 dstv0, dstv1, dstv2, dstv3, dstv4,
                 rows0, rows1, rows2, rows3, rows4, acc,
                 psem0, psem1, gsem0, gsem1, gsem2, gsem3, gsem4,
                 dsem0, dsem1, dsem2, dsem3, dsem4,
                 ssem0, ssem1, ssem2, ssem3, ssem4):
    srcp = (srcp0, srcp1)
    ewp = (ewp0, ewp1)
    psem = (psem0, psem1)
    dstv = (dstv0, dstv1, dstv2, dstv3, dstv4)
    rows = (rows0, rows1, rows2, rows3, rows4)
    gsem = (gsem0, gsem1, gsem2, gsem3, gsem4)
    dsem = (dsem0, dsem1, dsem2, dsem3, dsem4)
    ssem = (ssem0, ssem1, ssem2, ssem3, ssem4)
    c = lax.axis_index("c")
    s = lax.axis_index("s")
    wid = c * NS + s

    @plsc.parallel_loop(0, CHUNK, unroll=5)
    def _z(j):
        for r in range(D // L):
            rows[0][j, pl.ds(r * L, L)] = jnp.zeros((L,), jnp.float32)
    base_rows = s * RPT
    for m in range(SPAN // 40):
        pltpu.sync_copy(rows[0].at[pl.ds(0, 40)],
                        acc.at[pl.ds(base_rows + m * 40, 40)])
    plsc.subcore_barrier()

    def issue_pair(P, pb):
        pltpu.async_copy(src_hbm.at[wid, P, 0], srcp[pb], psem[pb])
        pltpu.async_copy(ew_hbm.at[wid, P, 0], ewp[pb], psem[pb])

    def wait_pair(P, pb):
        pltpu.make_async_copy(src_hbm.at[wid, P, 0], srcp[pb], psem[pb]).wait()
        pltpu.make_async_copy(ew_hbm.at[wid, P, 0], ewp[pb], psem[pb]).wait()

    def issue_dst_b(g, b):
        pltpu.async_copy(dst_hbm.at[wid, g, 0], dstv[b], dsem[b])

    def wait_dst(b, g):
        pltpu.make_async_copy(dst_hbm.at[wid, g, 0], dstv[b], dsem[b]).wait()

    def issue_gather(pb, lc, b):
        pltpu.async_copy(h_hbm.at[srcp[pb].at[pl.ds(lc * CHUNK, CHUNK)]],
                         rows[b], gsem[b])

    def wait_gather(pb, lc, b):
        pltpu.make_async_copy(h_hbm.at[srcp[pb].at[pl.ds(lc * CHUNK, CHUNK)]],
                              rows[b], gsem[b]).wait()

    def issue_scatter(b):
        pltpu.async_copy(rows[b], acc.at[dstv[b]], ssem[b], add=True)

    def wait_scatter(b):
        pltpu.make_async_copy(rows[b], acc.at[dstv[b]], ssem[b]).wait()

    def do_chunk(P, pb, grp, cc):
        lc = grp * NBUF + cc
        g = P * PAIR_C + lc
        b2 = (cc + 2) % NBUF

        if P == 0 and cc < 3:
            @pl.when(grp >= 1)
            def _():
                wait_scatter(b2)
        else:
            wait_scatter(b2)

        if cc < 3:
            issue_gather(pb, lc + 2, b2)
        else:
            @pl.when(grp < GRPS - 1)
            def _():
                issue_gather(pb, lc + 2, b2)

        if P == PAIRS - 1 and cc >= 3:
            @pl.when(grp < GRPS - 1)
            def _():
                issue_dst_b(g + 2, b2)
        else:
            issue_dst_b(g + 2, b2)

        wait_gather(pb, lc, cc)
        wait_dst(cc, g)

        @plsc.parallel_loop(0, CHUNK, unroll=5)
        def _scale(j):
            w = plsc.load_gather(ewp[pb], [jnp.broadcast_to(lc * CHUNK + j, (L,))])
            for r in range(D // L):
                rows[cc][j, pl.ds(r * L, L)] = rows[cc][j, pl.ds(r * L, L)] * w

        issue_scatter(cc)

    issue_pair(0, 0)
    wait_pair(0, 0)
    issue_pair(1, 1)
    issue_gather(0, 0, 0)
    issue_gather(0, 1, 1)
    issue_dst_b(jnp.int32(0), 0)
    issue_dst_b(jnp.int32(1), 1)

    for P in range(PAIRS):
        pb = P % 2

        def _grp_body(grp, _):
            for cc in range(NBUF):
                do_chunk(P, pb, grp, cc)
            return 0
        lax.fori_loop(0, GRPS, _grp_body, 0)

        if P < PAIRS - 1:
            npb = (P + 1) % 2
            wait_pair(P + 1, npb)
            if P < PAIRS - 2:
                issue_pair(P + 2, pb)
            issue_gather(npb, 0, 0)
            issue_gather(npb, 1, 1)

    for b in (2, 3, 4):
        wait_scatter(b)
    plsc.subcore_barrier()

    @pl.when(c == 0)
    def _():
        pltpu.sync_copy(acc.at[pl.ds(base_rows, SPAN)],
                        out0_hbm.at[pl.ds(base_rows, SPAN)])

    @pl.when(c == 1)
    def _():
        pltpu.sync_copy(acc.at[pl.ds(base_rows, SPAN)],
                        out1_hbm.at[pl.ds(base_rows, SPAN)])


@jax.jit
def _sc_aggregate(h, src_r, dst_r, ew_r):
    mesh = plsc.VectorSubcoreMesh(core_axis_name="c", subcore_axis_name="s")
    return pl.kernel(
        _sc_agg_body,
        out_type=(jax.ShapeDtypeStruct((N, D), jnp.float32),
                  jax.ShapeDtypeStruct((N, D), jnp.float32)),
        mesh=mesh,
        compiler_params=pltpu.CompilerParams(needs_layout_passes=False),
        scratch_types=(
            [pltpu.VMEM((PAIR_E,), jnp.int32)] * 2
            + [pltpu.VMEM((PAIR_E,), jnp.float32)] * 2
            + [pltpu.VMEM((CHUNK,), jnp.int32)] * NBUF
            + [pltpu.VMEM((CHUNK, D), jnp.float32)] * NBUF
            + [pltpu.VMEM_SHARED((N, D), jnp.float32)]
            + [pltpu.SemaphoreType.DMA] * (2 + 3 * NBUF)
        ),
    )(h, src_r, dst_r, ew_r)


BLK = 1000
NBLK = N // BLK


def _mlp_body(h_ref, a0_ref, a1_ref, w1_ref, b1_ref, w2_ref, b2_ref,
              z_ref, st_ref):
    z = h_ref[...] + a0_ref[...] + a1_ref[...]
    y = jnp.maximum(
        jnp.dot(z, w1_ref[...], preferred_element_type=jnp.float32) + b1_ref[...], 0.0)
    z2 = jnp.dot(y, w2_ref[...], preferred_element_type=jnp.float32) + b2_ref[...]
    z_ref[...] = z2
    s1 = jnp.sum(z2, axis=0, keepdims=True)
    s2 = jnp.sum(z2 * z2, axis=0, keepdims=True)
    st = jnp.concatenate([s1, s2, jnp.zeros((6, D), jnp.float32)], axis=0)

    @pl.when(pl.program_id(0) == 0)
    def _():
        st_ref[...] = jnp.zeros_like(st_ref)

    st_ref[...] += st


def _bn_body(z_ref, st_ref, g_ref, b_ref, o_ref):
    s = st_ref[...]
    mean = s[0:1] * (1.0 / N)
    var = s[1:2] * (1.0 / N) - mean * mean
    inv = lax.rsqrt(var + 1e-3)
    o_ref[...] = jnp.maximum(
        g_ref[...] * (z_ref[...] - mean) * inv + b_ref[...], 0.0)


def _head_body(h_ref, wm1_ref, bm1_ref, wm2_ref, bm2_ref, o_ref):
    y = jnp.maximum(
        jnp.dot(h_ref[...], wm1_ref[...], preferred_element_type=jnp.float32)
        + bm1_ref[...], 0.0)
    o_ref[...] = jnp.dot(y, wm2_ref[...], preferred_element_type=jnp.float32) + bm2_ref[...]


def _row_spec(width):
    return pl.BlockSpec((BLK, width), lambda i: (i, 0))


def _full_spec(shape):
    return pl.BlockSpec(shape, lambda i: tuple(0 for _ in shape))


@jax.jit
def _tc_mlp(h, a0, a1, w1, b1, w2, b2, gamma, beta):
    z, st = pl.pallas_call(
        _mlp_body,
        grid=(NBLK,),
        in_specs=[
            _row_spec(D), _row_spec(D), _row_spec(D),
            _full_spec((D, D)), _full_spec((1, D)),
            _full_spec((D, D)), _full_spec((1, D)),
        ],
        out_specs=[_row_spec(D), _full_spec((8, D))],
        out_shape=[
            jax.ShapeDtypeStruct((N, D), jnp.float32),
            jax.ShapeDtypeStruct((8, D), jnp.float32),
        ],
    )(h, a0, a1, w1, b1.reshape(1, D), w2, b2.reshape(1, D))
    return pl.pallas_call(
        _bn_body,
        grid=(NBLK,),
        in_specs=[
            _row_spec(D), _full_spec((8, D)),
            _full_spec((1, D)), _full_spec((1, D)),
        ],
        out_specs=_row_spec(D),
        out_shape=jax.ShapeDtypeStruct((N, D), jnp.float32),
    )(z, st, gamma.reshape(1, D), beta.reshape(1, D))


@jax.jit
def _tc_head(h, wm1, bm1, wm2, bm2):
    wm2p = jnp.zeros((256, 128), jnp.float32).at[:, :wm2.shape[1]].set(wm2)
    bm2p = jnp.zeros((1, 128), jnp.float32).at[:, :wm2.shape[1]].set(bm2.reshape(1, -1))
    out = pl.pallas_call(
        _head_body,
        grid=(NBLK,),
        in_specs=[
            _row_spec(D), _full_spec((D, 256)), _full_spec((1, 256)),
            _full_spec((256, 128)), _full_spec((1, 128)),
        ],
        out_specs=_row_spec(128),
        out_shape=jax.ShapeDtypeStruct((N, 128), jnp.float32),
    )(h, wm1, bm1.reshape(1, 256), wm2p, bm2p)
    return out[:, :wm2.shape[1]]


def kernel(x, edge_index, edge_weight, l0_W1, l0_b1, l0_W2, l0_b2, l0_gamma,
           l0_beta, l1_W1, l1_b1, l1_W2, l1_b2, l1_gamma, l1_beta,
           Wm1, bm1, Wm2, bm2):
    src_r = edge_index[0].reshape(NW, PAIRS, 1, PAIR_E)
    dst_r = edge_index[1].reshape(NW, NCHUNK, 1, CHUNK)
    ew_r = edge_weight.reshape(NW, PAIRS, 1, PAIR_E)
    h = x
    a0, a1 = _sc_aggregate(h, src_r, dst_r, ew_r)
    h = _tc_mlp(h, a0, a1, l0_W1, l0_b1, l0_W2, l0_b2, l0_gamma, l0_beta)
    a0, a1 = _sc_aggregate(h, src_r, dst_r, ew_r)
    h = _tc_mlp(h, a0, a1, l1_W1, l1_b1, l1_W2, l1_b2, l1_gamma, l1_beta)
    return _tc_head(h, Wm1, bm1, Wm2, bm2)


# speedup vs baseline: 9.7729x; 1.0202x over previous
import functools
import jax
import jax.numpy as jnp
from jax import lax
from jax.experimental import pallas as pl
from jax.experimental.pallas import tpu as pltpu
from jax.experimental.pallas import tpu_sc as plsc

N = 10000
E = 320000
D = 128
L = 16
NC = 2
NS = 16
NW = NC * NS
EPW = E // NW
CHUNK = 40
NCHUNK = EPW // CHUNK
NBUF = 5
PAIRS = 5
PAIR_E = EPW // PAIRS
PAIR_C = NCHUNK // PAIRS
GRPS = PAIR_C // NBUF
RPT = 624
SPAN = N - (NS - 1) * RPT


def _sc_agg_body(h_hbm, src_hbm, dst_hbm, ew_hbm, out0_hbm, out1_hbm,
                 srcp0, srcp1, ewp0, ewp1,
                 dstv0, dstv1, dstv2, dstv3, dstv4,
                 rows0, rows1, rows2, rows3, rows4, acc,
                 psem0, psem1, gsem0, gsem1, gsem2, gsem3, gsem4,
                 dsem0, dsem1, dsem2, dsem3, dsem4,
                 ssem0, ssem1, ssem2, ssem3, ssem4):
    srcp = (srcp0, srcp1)
    ewp = (ewp0, ewp1)
    psem = (psem0, psem1)
    dstv = (dstv0, dstv1, dstv2, dstv3, dstv4)
    rows = (rows0, rows1, rows2, rows3, rows4)
    gsem = (gsem0, gsem1, gsem2, gsem3, gsem4)
    dsem = (dsem0, dsem1, dsem2, dsem3, dsem4)
    ssem = (ssem0, ssem1, ssem2, ssem3, ssem4)
    c = lax.axis_index("c")
    s = lax.axis_index("s")
    wid = c * NS + s

    @plsc.parallel_loop(0, CHUNK, unroll=5)
    def _z(j):
        for r in range(D // L):
            rows[0][j, pl.ds(r * L, L)] = jnp.zeros((L,), jnp.float32)
    base_rows = s * RPT
    for m in range(SPAN // 40):
        pltpu.sync_copy(rows[0].at[pl.ds(0, 40)],
                        acc.at[pl.ds(base_rows + m * 40, 40)])
    plsc.subcore_barrier()

    def issue_pair(P, pb):
        pltpu.async_copy(src_hbm.at[wid, P, 0], srcp[pb], psem[pb])
        pltpu.async_copy(ew_hbm.at[wid, P, 0], ewp[pb], psem[pb])

    def wait_pair(P, pb):
        pltpu.make_async_copy(src_hbm.at[wid, P, 0], srcp[pb], psem[pb]).wait()
        pltpu.make_async_copy(ew_hbm.at[wid, P, 0], ewp[pb], psem[pb]).wait()

    def issue_dst_b(g, b):
        pltpu.async_copy(dst_hbm.at[wid, g, 0], dstv[b], dsem[b])

    def wait_dst(b, g):
        pltpu.make_async_copy(dst_hbm.at[wid, g, 0], dstv[b], dsem[b]).wait()

    def issue_gather(pb, lc, b):
        pltpu.async_copy(h_hbm.at[srcp[pb].at[pl.ds(lc * CHUNK, CHUNK)]],
                         rows[b], gsem[b])

    def wait_gather(pb, lc, b):
        pltpu.make_async_copy(h_hbm.at[srcp[pb].at[pl.ds(lc * CHUNK, CHUNK)]],
                              rows[b], gsem[b]).wait()

    def issue_scatter(b):
        pltpu.async_copy(rows[b], acc.at[dstv[b]], ssem[b], add=True)

    def wait_scatter(b):
        pltpu.make_async_copy(rows[b], acc.at[dstv[b]], ssem[b]).wait()

    def do_chunk(P, pb, grp, cc):
        lc = grp * NBUF + cc
        g = P * PAIR_C + lc
        b2 = (cc + 2) % NBUF

        if P == 0 and cc < 3:
            @pl.when(grp >= 1)
            def _():
                wait_scatter(b2)
        else:
            wait_scatter(b2)

        if cc < 3:
            issue_gather(pb, lc + 2, b2)
        else:
            @pl.when(grp < GRPS - 1)
            def _():
                issue_gather(pb, lc + 2, b2)

        if P == PAIRS - 1 and cc >= 3:
            @pl.when(grp < GRPS - 1)
            def _():
                issue_dst_b(g + 2, b2)
        else:
            issue_dst_b(g + 2, b2)

        wait_gather(pb, lc, cc)
        wait_dst(cc, g)

        @plsc.parallel_loop(0, CHUNK, unroll=5)
        def _scale(j):
            w = plsc.load_gather(ewp[pb], [jnp.broadcast_to(lc * CHUNK + j, (L,))])
            for r in range(D // L):
                rows[cc][j, pl.ds(r * L, L)] = rows[cc][j, pl.ds(r * L, L)] * w

        issue_scatter(cc)

    issue_pair(0, 0)
    wait_pair(0, 0)
    issue_pair(1, 1)
    issue_gather(0, 0, 0)
    issue_gather(0, 1, 1)
    issue_dst_b(jnp.int32(0), 0)
    issue_dst_b(jnp.int32(1), 1)

    for P in range(PAIRS):
        pb = P % 2

        def _grp_body(grp, _):
            for cc in range(NBUF):
                do_chunk(P, pb, grp, cc)
            return 0
        lax.fori_loop(0, GRPS, _grp_body, 0)

        if P < PAIRS - 1:
            npb = (P + 1) % 2
            wait_pair(P + 1, npb)
            if P < PAIRS - 2:
                issue_pair(P + 2, pb)
            issue_gather(npb, 0, 0)
            issue_gather(npb, 1, 1)

    for b in (2, 3, 4):
        wait_scatter(b)
    plsc.subcore_barrier()

    @pl.when(c == 0)
    def _():
        pltpu.sync_copy(acc.at[pl.ds(base_rows, SPAN)],
                        out0_hbm.at[pl.ds(base_rows, SPAN)])

    @pl.when(c == 1)
    def _():
        pltpu.sync_copy(acc.at[pl.ds(base_rows, SPAN)],
                        out1_hbm.at[pl.ds(base_rows, SPAN)])


@jax.jit
def _sc_aggregate(h, src_r, dst_r, ew_r):
    mesh = plsc.VectorSubcoreMesh(core_axis_name="c", subcore_axis_name="s")
    return pl.kernel(
        _sc_agg_body,
        out_type=(jax.ShapeDtypeStruct((N, D), jnp.float32),
                  jax.ShapeDtypeStruct((N, D), jnp.float32)),
        mesh=mesh,
        compiler_params=pltpu.CompilerParams(needs_layout_passes=False),
        scratch_types=(
            [pltpu.VMEM((PAIR_E,), jnp.int32)] * 2
            + [pltpu.VMEM((PAIR_E,), jnp.float32)] * 2
            + [pltpu.VMEM((CHUNK,), jnp.int32)] * NBUF
            + [pltpu.VMEM((CHUNK, D), jnp.float32)] * NBUF
            + [pltpu.VMEM_SHARED((N, D), jnp.float32)]
            + [pltpu.SemaphoreType.DMA] * (2 + 3 * NBUF)
        ),
    )(h, src_r, dst_r, ew_r)


BLK = 1000
NBLK = N // BLK


def _layer_phase0(i, h_ref, a0_ref, a1_ref, w1_ref, b1_ref, w2_ref, b2_ref,
                  zbuf, st):
    z = h_ref[...] + a0_ref[...] + a1_ref[...]
    y = jnp.maximum(
        jnp.dot(z, w1_ref[...], preferred_element_type=jnp.float32) + b1_ref[...], 0.0)
    z2 = jnp.dot(y, w2_ref[...], preferred_element_type=jnp.float32) + b2_ref[...]
    zbuf[pl.ds(i * BLK, BLK), :] = z2
    s1 = jnp.sum(z2, axis=0, keepdims=True)
    s2 = jnp.sum(z2 * z2, axis=0, keepdims=True)
    stv = jnp.concatenate([s1, s2, jnp.zeros((6, D), jnp.float32)], axis=0)

    @pl.when(i == 0)
    def _():
        st[...] = jnp.zeros_like(st)

    st[...] += stv


def _bn_block(i, g_ref, be_ref, zbuf, st):
    s = st[...]
    mean = s[0:1] * (1.0 / N)
    var = s[1:2] * (1.0 / N) - mean * mean
    inv = lax.rsqrt(var + 1e-3)
    return jnp.maximum(
        g_ref[...] * (zbuf[pl.ds(i * BLK, BLK), :] - mean) * inv + be_ref[...], 0.0)


def _mlp_fused_body(h_ref, a0_ref, a1_ref, w1_ref, b1_ref, w2_ref, b2_ref,
                    g_ref, be_ref, o_ref, zbuf, st):
    p = pl.program_id(0)
    i = pl.program_id(1)

    @pl.when(p == 0)
    def _():
        _layer_phase0(i, h_ref, a0_ref, a1_ref, w1_ref, b1_ref, w2_ref, b2_ref,
                      zbuf, st)

    @pl.when(p == 1)
    def _():
        o_ref[...] = _bn_block(i, g_ref, be_ref, zbuf, st)


def _mlp_head_body(h_ref, a0_ref, a1_ref, w1_ref, b1_ref, w2_ref, b2_ref,
                   g_ref, be_ref, wm1_ref, bm1_ref, wm2_ref, bm2_ref,
                   o_ref, zbuf, st):
    p = pl.program_id(0)
    i = pl.program_id(1)

    @pl.when(p == 0)
    def _():
        _layer_phase0(i, h_ref, a0_ref, a1_ref, w1_ref, b1_ref, w2_ref, b2_ref,
                      zbuf, st)

    @pl.when(p == 1)
    def _():
        hb = _bn_block(i, g_ref, be_ref, zbuf, st)
        y = jnp.maximum(
            jnp.dot(hb, wm1_ref[...], preferred_element_type=jnp.float32)
            + bm1_ref[...], 0.0)
        o_ref[...] = (jnp.dot(y, wm2_ref[...], preferred_element_type=jnp.float32)
                      + bm2_ref[...])


def _row_spec(width):
    return pl.BlockSpec((BLK, width), lambda p, i: (i, 0))


def _full_spec(shape):
    return pl.BlockSpec(shape, lambda p, i: tuple(0 for _ in shape))


_SCRATCH = [pltpu.VMEM((N, D), jnp.float32), pltpu.VMEM((8, D), jnp.float32)]
_LAYER_SPECS = [
    _row_spec(D), _row_spec(D), _row_spec(D),
    _full_spec((D, D)), _full_spec((1, D)),
    _full_spec((D, D)), _full_spec((1, D)),
    _full_spec((1, D)), _full_spec((1, D)),
]


@jax.jit
def _tc_mlp(h, a0, a1, w1, b1, w2, b2, gamma, beta):
    return pl.pallas_call(
        _mlp_fused_body,
        grid=(2, NBLK),
        in_specs=_LAYER_SPECS,
        out_specs=_row_spec(D),
        out_shape=jax.ShapeDtypeStruct((N, D), jnp.float32),
        scratch_shapes=_SCRATCH,
    )(h, a0, a1, w1, b1.reshape(1, D), w2, b2.reshape(1, D),
      gamma.reshape(1, D), beta.reshape(1, D))


@jax.jit
def _tc_mlp_head(h, a0, a1, w1, b1, w2, b2, gamma, beta, wm1, bm1, wm2, bm2):
    wm2p = jnp.zeros((256, 128), jnp.float32).at[:, :wm2.shape[1]].set(wm2)
    bm2p = jnp.zeros((1, 128), jnp.float32).at[:, :wm2.shape[1]].set(bm2.reshape(1, -1))
    out = pl.pallas_call(
        _mlp_head_body,
        grid=(2, NBLK),
        in_specs=_LAYER_SPECS + [
            _full_spec((D, 256)), _full_spec((1, 256)),
            _full_spec((256, 128)), _full_spec((1, 128)),
        ],
        out_specs=_row_spec(128),
        out_shape=jax.ShapeDtypeStruct((N, 128), jnp.float32),
        scratch_shapes=_SCRATCH,
    )(h, a0, a1, w1, b1.reshape(1, D), w2, b2.reshape(1, D),
      gamma.reshape(1, D), beta.reshape(1, D),
      wm1, bm1.reshape(1, 256), wm2p, bm2p)
    return out[:, :wm2.shape[1]]


def kernel(x, edge_index, edge_weight, l0_W1, l0_b1, l0_W2, l0_b2, l0_gamma,
           l0_beta, l1_W1, l1_b1, l1_W2, l1_b2, l1_gamma, l1_beta,
           Wm1, bm1, Wm2, bm2):
    src_r = edge_index[0].reshape(NW, PAIRS, 1, PAIR_E)
    dst_r = edge_index[1].reshape(NW, NCHUNK, 1, CHUNK)
    ew_r = edge_weight.reshape(NW, PAIRS, 1, PAIR_E)
    h = x
    a0, a1 = _sc_aggregate(h, src_r, dst_r, ew_r)
    h = _tc_mlp(h, a0, a1, l0_W1, l0_b1, l0_W2, l0_b2, l0_gamma, l0_beta)
    a0, a1 = _sc_aggregate(h, src_r, dst_r, ew_r)
    return _tc_mlp_head(h, a0, a1, l1_W1, l1_b1, l1_W2, l1_b2, l1_gamma,
                        l1_beta, Wm1, bm1, Wm2, bm2)
